# Initial kernel scaffold; baseline (speedup 1.0000x reference)
import functools

import jax
import jax.numpy as jnp
from jax import lax
from jax.experimental import pallas as pl
from jax.experimental.pallas import tpu as pltpu
from jax.experimental.pallas import tpu_sc as plsc

N_USERS = 50000
N_ITEMS = 50000
N = N_USERS + N_ITEMS
DIM = 32
H = DIM // 2
E = 1600000
B = 4096
N_LAYERS = 3

NC = 2
NS = 16
LANES = 16

CH = 2000
EDGES_PER_SUB = E // NS
CHUNKS = EDGES_PER_SUB // CH
ROWS_PER_SUB = N // NS
ZCOPIES = ROWS_PER_SUB // CH
ZTAIL = ROWS_PER_SUB - ZCOPIES * CH

_MESH = plsc.VectorSubcoreMesh(core_axis_name="c", subcore_axis_name="s")

_IDX16 = [jnp.full((LANES,), t, dtype=jnp.int32) for t in range(LANES)]


def _scale_rows(rows_v, ev_v):
    @pl.loop(0, CH, step=LANES)
    def _(g):
        ev16 = ev_v[pl.ds(g, LANES)]
        for t in range(LANES):
            bt = ev16.at[_IDX16[t]].get(mode="promise_in_bounds")
            rows_v[g + t, :] = rows_v[g + t, :] * bt


@functools.partial(
    pl.kernel,
    out_type=(
        jax.ShapeDtypeStruct((N, H), jnp.float32),
        jax.ShapeDtypeStruct((N, H), jnp.float32),
    ),
    mesh=_MESH,
    scratch_types=[
        pltpu.VMEM((CH,), jnp.int32),
        pltpu.VMEM((CH,), jnp.int32),
        pltpu.VMEM((CH,), jnp.float32),
        pltpu.VMEM((CH, H), jnp.float32),
        pltpu.VMEM_SHARED((N, H), jnp.float32),
    ],
)
def _layer(tl, tr, src, dst, ev, out_l, out_r, src_v, dst_v, ev_v, rows_v, acc):
    c = lax.axis_index("c")
    s = lax.axis_index("s")

    @pl.loop(0, CH)
    def _(i):
        rows_v[i, :] = jnp.zeros((LANES,), jnp.float32)

    r0 = s * ROWS_PER_SUB
    for k in range(ZCOPIES):
        pltpu.sync_copy(rows_v, acc.at[pl.ds(r0 + k * CH, CH)])
    pltpu.sync_copy(rows_v.at[pl.ds(0, ZTAIL)], acc.at[pl.ds(r0 + ZCOPIES * CH, ZTAIL)])
    plsc.subcore_barrier()

    def do_core(table_ref, out_ref):
        base_e = s * EDGES_PER_SUB

        @pl.loop(0, CHUNKS)
        def _(ci):
            e0 = base_e + ci * CH
            pltpu.sync_copy(src.at[pl.ds(e0, CH)], src_v)
            pltpu.sync_copy(dst.at[pl.ds(e0, CH)], dst_v)
            pltpu.sync_copy(ev.at[pl.ds(e0, CH)], ev_v)
            pltpu.sync_copy(table_ref.at[src_v], rows_v)
            _scale_rows(rows_v, ev_v)
            pltpu.sync_copy(rows_v, acc.at[dst_v], add=True)

        plsc.subcore_barrier()
        for k in range(ZCOPIES):
            pltpu.sync_copy(acc.at[pl.ds(r0 + k * CH, CH)], out_ref.at[pl.ds(r0 + k * CH, CH)])
        pltpu.sync_copy(
            acc.at[pl.ds(r0 + ZCOPIES * CH, ZTAIL)],
            out_ref.at[pl.ds(r0 + ZCOPIES * CH, ZTAIL)],
        )

    @pl.when(c == 0)
    def _():
        do_core(tl, out_l)

    @pl.when(c == 1)
    def _():
        do_core(tr, out_r)


NB = 3 * B
RPS = NB // NS


@functools.partial(
    pl.kernel,
    out_type=(
        jax.ShapeDtypeStruct((NB, H), jnp.float32),
        jax.ShapeDtypeStruct((NB, H), jnp.float32),
        jax.ShapeDtypeStruct((NB, H), jnp.float32),
        jax.ShapeDtypeStruct((NB, H), jnp.float32),
    ),
    mesh=_MESH,
    scratch_types=[
        pltpu.VMEM((RPS,), jnp.int32),
        pltpu.VMEM((RPS, H), jnp.float32),
        pltpu.VMEM((RPS, H), jnp.float32),
        pltpu.VMEM((RPS, H), jnp.float32),
        pltpu.VMEM((RPS, H), jnp.float32),
    ],
)
def _final(t0l, t0r, t1l, t1r, t2l, t2r, t3l, t3r, idx,
           mean_l, mean_r, init_l, init_r, idx_v, g0, g1, g2, g3):
    c = lax.axis_index("c")
    s = lax.axis_index("s")
    base = s * RPS
    pltpu.sync_copy(idx.at[pl.ds(base, RPS)], idx_v)

    def do_core(T0, T1, T2, T3, mean_out, init_out):
        pltpu.sync_copy(T0.at[idx_v], g0)
        pltpu.sync_copy(T1.at[idx_v], g1)
        pltpu.sync_copy(T2.at[idx_v], g2)
        pltpu.sync_copy(T3.at[idx_v], g3)
        pltpu.sync_copy(g0, init_out.at[pl.ds(base, RPS)])

        @pl.loop(0, RPS)
        def _(i):
            m = (g0[i, :] + g1[i, :]) + (g2[i, :] + g3[i, :])
            g0[i, :] = m * jnp.float32(0.25)

        pltpu.sync_copy(g0, mean_out.at[pl.ds(base, RPS)])

    @pl.when(c == 0)
    def _():
        do_core(t0l, t1l, t2l, t3l, mean_l, init_l)

    @pl.when(c == 1)
    def _():
        do_core(t0r, t1r, t2r, t3r, mean_r, init_r)


def kernel(users, pos_items, neg_items, edge_index, edge_values, user_emb, item_emb):
    all0 = jnp.concatenate([user_emb, item_emb], axis=0)
    tl = jnp.ascontiguousarray(all0[:, :H])
    tr = jnp.ascontiguousarray(all0[:, H:])
    src = edge_index[0].astype(jnp.int32)
    dst = edge_index[1].astype(jnp.int32)
    ev = edge_values.astype(jnp.float32)

    tabs = [(tl, tr)]
    for _ in range(N_LAYERS):
        tl, tr = _layer(tl, tr, src, dst, ev)
        tabs.append((tl, tr))

    idx_all = jnp.concatenate([
        users.astype(jnp.int32),
        pos_items.astype(jnp.int32) + N_USERS,
        neg_items.astype(jnp.int32) + N_USERS,
    ])

    mean_l, mean_r, init_l, init_r = _final(
        tabs[0][0], tabs[0][1], tabs[1][0], tabs[1][1],
        tabs[2][0], tabs[2][1], tabs[3][0], tabs[3][1], idx_all,
    )

    mean = jnp.concatenate([mean_l, mean_r], axis=1)
    init = jnp.concatenate([init_l, init_r], axis=1)
    return (
        mean[:B], mean[B:2 * B], mean[2 * B:],
        init[:B], init[B:2 * B], init[2 * B:],
    )


# speedup vs baseline: 7.3748x; 7.3748x over previous
import dataclasses
import functools

import numpy as np
import jax
import jax.numpy as jnp
from jax import lax
from jax.experimental import pallas as pl
from jax.experimental.pallas import tpu as pltpu
from jax.experimental.pallas import tpu_sc as plsc

N_USERS = 50000
N_ITEMS = 50000
N = N_USERS + N_ITEMS
DIM = 32
H = DIM // 2
E = 1600000
B = 4096
N_LAYERS = 3

NC = 2
NS = 16
LANES = 16

CH = 1000
EDGES_PER_SUB = E // NS
CHUNKS = EDGES_PER_SUB // CH
NCHUNKS_N = N // CH

_MESH = plsc.VectorSubcoreMesh(core_axis_name="c", subcore_axis_name="s")

_CP = pltpu.CompilerParams(
    needs_layout_passes=False,
    use_tc_tiling_on_sc=False,
)

def _scale_rows(rows_v, ev_v):
    zero16 = lax.iota(jnp.int32, LANES) * 0

    @pl.loop(0, CH, step=LANES)
    def _(g):
        for t in range(LANES):
            bt = plsc.load_gather(ev_v, [zero16 + (g + t)])
            rows_v[g + t, :] = rows_v[g + t, :] * bt


@functools.partial(
    pl.kernel,
    out_type=(
        jax.ShapeDtypeStruct((N, H), jnp.float32),
        jax.ShapeDtypeStruct((N, H), jnp.float32),
    ),
    mesh=_MESH,
    scratch_types=[
        pltpu.VMEM((CH,), jnp.int32),
        pltpu.VMEM((CH,), jnp.int32),
        pltpu.VMEM((CH,), jnp.float32),
        pltpu.VMEM((CH, H), jnp.float32),
        pltpu.VMEM_SHARED((N, H), jnp.float32),
    ],
    compiler_params=_CP,
)
def _layer(tl, tr, src, dst, ev, out_l, out_r, src_v, dst_v, ev_v, rows_v, acc):
    c = lax.axis_index("c")
    s = lax.axis_index("s")

    k_lo = (s * NCHUNKS_N) // NS
    k_hi = ((s + 1) * NCHUNKS_N) // NS

    @pl.loop(0, CH)
    def _(i):
        rows_v[i, :] = jnp.zeros((LANES,), jnp.float32)

    @pl.loop(k_lo, k_hi)
    def _(k):
        pltpu.sync_copy(rows_v, acc.at[pl.ds(k * CH, CH)])

    plsc.subcore_barrier()

    def do_core(table_ref, out_ref):
        base_e = s * EDGES_PER_SUB

        @pl.loop(0, CHUNKS)
        def _(ci):
            e0 = base_e + ci * CH
            pltpu.sync_copy(src.at[pl.ds(e0, CH)], src_v)
            pltpu.sync_copy(dst.at[pl.ds(e0, CH)], dst_v)
            pltpu.sync_copy(ev.at[pl.ds(e0, CH)], ev_v)
            pltpu.sync_copy(table_ref.at[src_v], rows_v)
            _scale_rows(rows_v, ev_v)
            pltpu.sync_copy(rows_v, acc.at[dst_v], add=True)

        plsc.subcore_barrier()

        @pl.loop(k_lo, k_hi)
        def _(k):
            pltpu.sync_copy(acc.at[pl.ds(k * CH, CH)], out_ref.at[pl.ds(k * CH, CH)])

    @pl.when(c == 0)
    def _():
        do_core(tl, out_l)

    @pl.when(c == 1)
    def _():
        do_core(tr, out_r)


NB = 3 * B
RPS = NB // NS


@functools.partial(
    pl.kernel,
    out_type=(
        jax.ShapeDtypeStruct((NB, H), jnp.float32),
        jax.ShapeDtypeStruct((NB, H), jnp.float32),
        jax.ShapeDtypeStruct((NB, H), jnp.float32),
        jax.ShapeDtypeStruct((NB, H), jnp.float32),
    ),
    mesh=_MESH,
    scratch_types=[
        pltpu.VMEM((RPS,), jnp.int32),
        pltpu.VMEM((RPS, H), jnp.float32),
        pltpu.VMEM((RPS, H), jnp.float32),
        pltpu.VMEM((RPS, H), jnp.float32),
        pltpu.VMEM((RPS, H), jnp.float32),
    ],
    compiler_params=_CP,
)
def _final(t0l, t0r, t1l, t1r, t2l, t2r, t3l, t3r, idx,
           mean_l, mean_r, init_l, init_r, idx_v, g0, g1, g2, g3):
    c = lax.axis_index("c")
    s = lax.axis_index("s")
    base = s * RPS
    pltpu.sync_copy(idx.at[pl.ds(base, RPS)], idx_v)

    def do_core(T0, T1, T2, T3, mean_out, init_out):
        pltpu.sync_copy(T0.at[idx_v], g0)
        pltpu.sync_copy(T1.at[idx_v], g1)
        pltpu.sync_copy(T2.at[idx_v], g2)
        pltpu.sync_copy(T3.at[idx_v], g3)
        pltpu.sync_copy(g0, init_out.at[pl.ds(base, RPS)])

        @pl.loop(0, RPS)
        def _(i):
            m = (g0[i, :] + g1[i, :]) + (g2[i, :] + g3[i, :])
            g0[i, :] = m * jnp.float32(0.25)

        pltpu.sync_copy(g0, mean_out.at[pl.ds(base, RPS)])

    @pl.when(c == 0)
    def _():
        do_core(t0l, t1l, t2l, t3l, mean_l, init_l)

    @pl.when(c == 1)
    def _():
        do_core(t0r, t1r, t2r, t3r, mean_r, init_r)


def kernel(users, pos_items, neg_items, edge_index, edge_values, user_emb, item_emb):
    all0 = jnp.concatenate([user_emb, item_emb], axis=0)
    tl = all0[:, :H] + jnp.float32(0.0)
    tr = all0[:, H:] + jnp.float32(0.0)
    src = edge_index[0].astype(jnp.int32)
    dst = edge_index[1].astype(jnp.int32)
    ev = edge_values.astype(jnp.float32)

    tabs = [(tl, tr)]
    for _ in range(N_LAYERS):
        tl, tr = _layer(tl, tr, src, dst, ev)
        tabs.append((tl, tr))

    idx_all = jnp.concatenate([
        users.astype(jnp.int32),
        pos_items.astype(jnp.int32) + N_USERS,
        neg_items.astype(jnp.int32) + N_USERS,
    ])

    mean_l, mean_r, init_l, init_r = _final(
        tabs[0][0], tabs[0][1], tabs[1][0], tabs[1][1],
        tabs[2][0], tabs[2][1], tabs[3][0], tabs[3][1], idx_all,
    )

    mean = jnp.concatenate([mean_l, mean_r], axis=1)
    init = jnp.concatenate([init_l, init_r], axis=1)
    return (
        mean[:B], mean[B:2 * B], mean[2 * B:],
        init[:B], init[B:2 * B], init[2 * B:],
    )
